# Initial kernel scaffold; baseline (speedup 1.0000x reference)
import jax
import jax.numpy as jnp
from jax.experimental import pallas as pl


def kernel(words_seq, table):
    raise NotImplementedError("write your pallas kernel here")


# speedup vs baseline: 5.1125x; 5.1125x over previous
import functools

import jax
import jax.numpy as jnp
from jax import lax
from jax.experimental import pallas as pl
from jax.experimental.pallas import tpu as pltpu
from jax.experimental.pallas import tpu_sc as plsc

VOCAB = 100000
EMBED_DIM = 32
BATCH = 4096
SEQ = 200

N = BATCH * SEQ
NC, NS = 2, 16
NW = NC * NS
PER_W = N // NW
CHUNK = 1280
NCH = PER_W // CHUNK

_MESH = plsc.VectorSubcoreMesh(
    core_axis_name="c", subcore_axis_name="s", num_cores=NC, num_subcores=NS
)


@functools.partial(
    pl.kernel,
    out_type=jax.ShapeDtypeStruct((N, EMBED_DIM), jnp.float32),
    mesh=_MESH,
    compiler_params=pltpu.CompilerParams(use_tc_tiling_on_sc=False),
    scratch_types=[
        pltpu.VMEM((CHUNK,), jnp.int32),
        pltpu.VMEM((CHUNK, EMBED_DIM), jnp.float32),
        pltpu.SemaphoreType.DMA,
    ],
)
def _gather_kernel(idx_hbm, table_hbm, out_hbm, idx_v, rows_v, sem):
    wid = lax.axis_index("s") * NC + lax.axis_index("c")
    base = wid * PER_W

    @pl.loop(0, NCH)
    def _chunk(i):
        pltpu.sync_copy(idx_hbm.at[wid, i], idx_v)
        pltpu.async_copy(table_hbm.at[idx_v], rows_v, sem).wait()
        pltpu.sync_copy(rows_v, out_hbm.at[pl.ds(base + i * CHUNK, CHUNK)])


def kernel(words_seq, table):
    idx = words_seq.reshape(NW, NCH, CHUNK)
    out = _gather_kernel(idx, table)
    return out.reshape(BATCH, SEQ, EMBED_DIM)


# speedup vs baseline: 5.2595x; 1.0287x over previous
import functools

import jax
import jax.numpy as jnp
from jax import lax
from jax.experimental import pallas as pl
from jax.experimental.pallas import tpu as pltpu
from jax.experimental.pallas import tpu_sc as plsc

VOCAB = 100000
EMBED_DIM = 32
BATCH = 4096
SEQ = 200

N = BATCH * SEQ
NC, NS = 2, 16
NW = NC * NS
PER_W = N // NW
CHUNK = 1280
NCH = PER_W // CHUNK

_MESH = plsc.VectorSubcoreMesh(
    core_axis_name="c", subcore_axis_name="s", num_cores=NC, num_subcores=NS
)


@functools.partial(
    pl.kernel,
    out_type=jax.ShapeDtypeStruct((N, EMBED_DIM), jnp.float32),
    mesh=_MESH,
    compiler_params=pltpu.CompilerParams(use_tc_tiling_on_sc=False),
    scratch_types=[
        pltpu.VMEM((CHUNK,), jnp.int32),
        pltpu.VMEM((CHUNK,), jnp.int32),
        pltpu.VMEM((CHUNK, EMBED_DIM), jnp.float32),
        pltpu.VMEM((CHUNK, EMBED_DIM), jnp.float32),
        pltpu.SemaphoreType.DMA,
        pltpu.SemaphoreType.DMA,
        pltpu.SemaphoreType.DMA,
        pltpu.SemaphoreType.DMA,
        pltpu.SemaphoreType.DMA,
        pltpu.SemaphoreType.DMA,
    ],
)
def _gather_kernel(idx_hbm, table_hbm, out_hbm, idx0, idx1, rows0, rows1,
                   is0, is1, gs0, gs1, ss0, ss1):
    wid = lax.axis_index("s") * NC + lax.axis_index("c")
    base = wid * PER_W
    idxb = (idx0, idx1)
    rows = (rows0, rows1)
    isem = (is0, is1)
    gsem = (gs0, gs1)
    ssem = (ss0, ss1)

    pltpu.async_copy(idx_hbm.at[wid, 0], idx0, is0)
    pltpu.async_copy(idx_hbm.at[wid, 1], idx1, is1)

    @pl.loop(0, NCH, step=2)
    def _outer(o):
        for b in range(2):
            i = o + b
            pltpu.make_async_copy(idx_hbm.at[wid, 0], idxb[b], isem[b]).wait()

            @pl.when(i >= 2)
            def _():
                pltpu.make_async_copy(
                    rows[b], out_hbm.at[pl.ds(base, CHUNK)], ssem[b]
                ).wait()

            pltpu.async_copy(table_hbm.at[idxb[b]], rows[b], gsem[b]).wait()

            @pl.when(i + 2 < NCH)
            def _():
                pltpu.async_copy(idx_hbm.at[wid, i + 2], idxb[b], isem[b])

            pltpu.async_copy(
                rows[b], out_hbm.at[pl.ds(base + i * CHUNK, CHUNK)], ssem[b]
            )

    pltpu.make_async_copy(rows0, out_hbm.at[pl.ds(base, CHUNK)], ss0).wait()
    pltpu.make_async_copy(rows1, out_hbm.at[pl.ds(base, CHUNK)], ss1).wait()


def kernel(words_seq, table):
    idx = words_seq.reshape(NW, NCH, CHUNK)
    out = _gather_kernel(idx, table)
    return out.reshape(BATCH, SEQ, EMBED_DIM)


# speedup vs baseline: 5.2898x; 1.0058x over previous
import functools

import jax
import jax.numpy as jnp
from jax import lax
from jax.experimental import pallas as pl
from jax.experimental.pallas import tpu as pltpu
from jax.experimental.pallas import tpu_sc as plsc

VOCAB = 100000
EMBED_DIM = 32
BATCH = 4096
SEQ = 200

N = BATCH * SEQ
NC, NS = 2, 16
NW = NC * NS
PER_W = N // NW
CHUNK = 400
G = 8
NCH = PER_W // CHUNK
NG = NCH // G

_MESH = plsc.VectorSubcoreMesh(
    core_axis_name="c", subcore_axis_name="s", num_cores=NC, num_subcores=NS
)


@functools.partial(
    pl.kernel,
    out_type=jax.ShapeDtypeStruct((N, EMBED_DIM), jnp.float32),
    mesh=_MESH,
    compiler_params=pltpu.CompilerParams(use_tc_tiling_on_sc=False),
    scratch_types=(
        [pltpu.VMEM((CHUNK,), jnp.int32) for _ in range(G)]
        + [pltpu.VMEM((CHUNK, EMBED_DIM), jnp.float32) for _ in range(G)]
        + [pltpu.SemaphoreType.DMA for _ in range(3 * G)]
    ),
)
def _gather_kernel(idx_hbm, table_hbm, out_hbm, *bufs):
    idxb = bufs[0:G]
    rows = bufs[G:2 * G]
    isem = bufs[2 * G:3 * G]
    gsem = bufs[3 * G:4 * G]
    ssem = bufs[4 * G:5 * G]

    wid = lax.axis_index("s") * NC + lax.axis_index("c")
    base = wid * PER_W

    for b in range(G):
        pltpu.async_copy(idx_hbm.at[wid, b], idxb[b], isem[b])

    @pl.loop(0, NG)
    def _group(g):
        descs = []
        for b in range(G):
            i = g * G + b
            pltpu.make_async_copy(idx_hbm.at[wid, 0], idxb[b], isem[b]).wait()

            @pl.when(g >= 1)
            def _():
                pltpu.make_async_copy(
                    rows[b], out_hbm.at[pl.ds(base, CHUNK)], ssem[b]
                ).wait()

            descs.append(pltpu.async_copy(table_hbm.at[idxb[b]], rows[b], gsem[b]))

        for b in range(G):
            i = g * G + b
            descs[b].wait()

            @pl.when(i + G < NCH)
            def _():
                pltpu.async_copy(idx_hbm.at[wid, i + G], idxb[b], isem[b])

            pltpu.async_copy(
                rows[b], out_hbm.at[pl.ds(base + i * CHUNK, CHUNK)], ssem[b]
            )

    for b in range(G):
        pltpu.make_async_copy(
            rows[b], out_hbm.at[pl.ds(base, CHUNK)], ssem[b]
        ).wait()


def kernel(words_seq, table):
    idx = words_seq.reshape(NW, NCH, CHUNK)
    out = _gather_kernel(idx, table)
    return out.reshape(BATCH, SEQ, EMBED_DIM)
